# baseline (speedup 1.0000x reference)
import functools

import jax
import jax.numpy as jnp
from jax import lax
from jax.experimental import pallas as pl
from jax.experimental.pallas import tpu as pltpu
from jax.experimental.pallas import tpu_sc as plsc

B, H, D = 16384, 50, 32
NW = 32
BW = B // NW
GW = 128
K = BW // GW
NPAIR = H // 2
RP = D + 1
TCW = BW // GW
OUT_MINOR = B * 8

_mesh = plsc.VectorSubcoreMesh(core_axis_name="c", subcore_axis_name="s")


@functools.partial(
    pl.kernel,
    mesh=_mesh,
    out_type=jax.ShapeDtypeStruct((H, D // 8, OUT_MINOR), jnp.float32),
    compiler_params=pltpu.CompilerParams(
        needs_layout_passes=False, use_tc_tiling_on_sc=False
    ),
    scratch_types=[
        pltpu.VMEM((K, GW), jnp.int32),
        pltpu.VMEM((K, GW), jnp.int32),
        pltpu.VMEM((BW, D), jnp.float32),
        pltpu.VMEM((BW, D), jnp.float32),
        pltpu.VMEM((D // 8, TCW * 1024), jnp.float32),
        pltpu.VMEM((D // 8, TCW * 1024), jnp.float32),
        pltpu.SemaphoreType.DMA,
        pltpu.SemaphoreType.DMA,
        pltpu.SemaphoreType.DMA,
        pltpu.SemaphoreType.DMA,
    ],
)
def _emb(idx_hbm, table_hbm, out_hbm, idx_v0, idx_v1, rows_v0, rows_v1,
         col_v0, col_v1, sem_g0, sem_g1, sem_o0, sem_o1):
    wid = lax.axis_index("s") * 2 + lax.axis_index("c")
    idx_v = (idx_v0, idx_v1)
    rows_v = (rows_v0, rows_v1)
    col_v = (col_v0, col_v1)
    sem_g = (sem_g0, sem_g1)
    sem_o = (sem_o0, sem_o1)
    iota16 = lax.iota(jnp.int32, 16)

    def fire(b, h):
        pltpu.sync_copy(idx_hbm.at[h, pl.ds(wid * K, K)], idx_v[b])
        for j in range(K):
            pltpu.async_copy(
                table_hbm.at[idx_v[b].at[j]],
                rows_v[b].at[pl.ds(j * GW, GW)],
                sem_g[b],
            )

    def drain_gathers(b):
        for j in range(K):
            pltpu.make_async_copy(
                table_hbm.at[idx_v[b].at[j]],
                rows_v[b].at[pl.ds(j * GW, GW)],
                sem_g[b],
            ).wait()

    def out_fire(b, h):
        for tr in range(D // 8):
            pltpu.async_copy(
                col_v[b].at[tr],
                out_hbm.at[h, tr, pl.ds(wid * (TCW * 1024), TCW * 1024)],
                sem_o[b],
            )

    def out_drain(b, h):
        for tr in range(D // 8):
            pltpu.make_async_copy(
                col_v[b].at[tr],
                out_hbm.at[h, tr, pl.ds(wid * (TCW * 1024), TCW * 1024)],
                sem_o[b],
            ).wait()

    def process(b, h):
        drain_gathers(b)

        def grp_body(g, carry2):
            vi = idx_v[b][g // (GW // 16), pl.ds((g % (GW // 16)) * 16, 16)]
            mb = vi != 0
            rowidx = g * 16 + iota16
            coff = (g // 8) * 1024 + (g % 8) * 16
            for d in range(D):
                val = plsc.load_gather(
                    rows_v[b], [rowidx, jnp.full((16,), d, jnp.int32)]
                )
                val = jnp.where(mb, val, jnp.float32(0.0))
                col_v[b][d // 8, pl.ds(coff + (d % 8) * 128, 16)] = val
            return carry2

        lax.fori_loop(0, BW // 16, grp_body, 0)

        out_fire(b, h)

    fire(0, 0)

    def pair_body(c2, carry):
        a = 2 * c2
        fire(1, a + 1)
        process(0, a)
        process(1, a + 1)
        out_drain(0, a)

        @pl.when(c2 < NPAIR - 1)
        def _():
            fire(0, a + 2)

        out_drain(1, a + 1)
        return carry

    lax.fori_loop(0, NPAIR, pair_body, 0)


def kernel(x, table):
    idx3d = jnp.transpose(x.astype(jnp.int32)).reshape(H, B // GW, GW)
    out_t = _emb(idx3d, table)
    o5 = out_t.reshape(H, D // 8, B // 128, 8, 128)
    return jnp.transpose(o5, (2, 4, 0, 1, 3)).reshape(B, H, D)


# speedup vs baseline: 1.2456x; 1.2456x over previous
import functools

import jax
import jax.numpy as jnp
from jax import lax
from jax.experimental import pallas as pl
from jax.experimental.pallas import tpu as pltpu
from jax.experimental.pallas import tpu_sc as plsc

B, H, D = 16384, 50, 32
NW = 32
BW = B // NW
GW = 128
K = BW // GW
NPAIR = H // 2

_mesh = plsc.VectorSubcoreMesh(core_axis_name="c", subcore_axis_name="s")


@functools.partial(
    pl.kernel,
    mesh=_mesh,
    out_type=jax.ShapeDtypeStruct((H, B, D), jnp.float32),
    compiler_params=pltpu.CompilerParams(
        needs_layout_passes=False, use_tc_tiling_on_sc=False
    ),
    scratch_types=[
        pltpu.VMEM((K, GW), jnp.int32),
        pltpu.VMEM((K, GW), jnp.int32),
        pltpu.VMEM((BW, D), jnp.float32),
        pltpu.VMEM((BW, D), jnp.float32),
        pltpu.VMEM((16,), jnp.float32),
        pltpu.SemaphoreType.DMA,
        pltpu.SemaphoreType.DMA,
        pltpu.SemaphoreType.DMA,
        pltpu.SemaphoreType.DMA,
    ],
)
def _emb(idx_hbm, table_hbm, out_hbm, idx_v0, idx_v1, rows_v0, rows_v1,
         mask_v, sem_g0, sem_g1, sem_o0, sem_o1):
    wid = lax.axis_index("s") * 2 + lax.axis_index("c")
    idx_v = (idx_v0, idx_v1)
    rows_v = (rows_v0, rows_v1)
    sem_g = (sem_g0, sem_g1)
    sem_o = (sem_o0, sem_o1)

    def fire(b, h):
        pltpu.sync_copy(idx_hbm.at[h, pl.ds(wid * K, K)], idx_v[b])
        for j in range(K):
            pltpu.async_copy(
                table_hbm.at[idx_v[b].at[j]],
                rows_v[b].at[pl.ds(j * GW, GW)],
                sem_g[b],
            )

    def drain_gathers(b):
        for j in range(K):
            pltpu.make_async_copy(
                table_hbm.at[idx_v[b].at[j]],
                rows_v[b].at[pl.ds(j * GW, GW)],
                sem_g[b],
            ).wait()

    def out_fire(b, h):
        pltpu.async_copy(
            rows_v[b], out_hbm.at[h, pl.ds(wid * BW, BW)], sem_o[b]
        )

    def out_drain(b, h):
        pltpu.make_async_copy(
            rows_v[b], out_hbm.at[h, pl.ds(wid * BW, BW)], sem_o[b]
        ).wait()

    def process(b, h):
        def det_body(j, acc):
            a = acc
            for t in range(GW // 16):
                v = idx_v[b][j, pl.ds(t * 16, 16)]
                a = a + jnp.where(v == 0, 1, 0).astype(jnp.int32)
            return a

        acc = lax.fori_loop(0, K, det_body, jnp.zeros((16,), jnp.int32))
        any_pad = jnp.sum(acc) > 0

        drain_gathers(b)

        @pl.when(any_pad)
        def _mask_pass():
            def grp_body(g, carry2):
                vi = idx_v[b][g // (GW // 16), pl.ds((g % (GW // 16)) * 16, 16)]
                mask_v[...] = jnp.where(vi == 0, 0.0, 1.0).astype(jnp.float32)
                for r in range(16):
                    em = plsc.load_gather(
                        mask_v, [jnp.full((16,), r, jnp.int32)]
                    )
                    row = g * 16 + r
                    for h2 in range(2):
                        cur = rows_v[b][row, pl.ds(h2 * 16, 16)]
                        rows_v[b][row, pl.ds(h2 * 16, 16)] = cur * em
                return carry2

            lax.fori_loop(0, BW // 16, grp_body, 0)

        out_fire(b, h)

    fire(0, 0)

    def pair_body(c2, carry):
        a = 2 * c2
        fire(1, a + 1)
        process(0, a)
        process(1, a + 1)
        out_drain(0, a)

        @pl.when(c2 < NPAIR - 1)
        def _():
            fire(0, a + 2)

        out_drain(1, a + 1)
        return carry

    lax.fori_loop(0, NPAIR, pair_body, 0)


def kernel(x, table):
    idx3d = jnp.transpose(x.astype(jnp.int32)).reshape(H, B // GW, GW)
    out_t = _emb(idx3d, table)
    return jnp.transpose(out_t, (1, 0, 2))


# speedup vs baseline: 1.2459x; 1.0003x over previous
import functools

import jax
import jax.numpy as jnp
from jax import lax
from jax.experimental import pallas as pl
from jax.experimental.pallas import tpu as pltpu
from jax.experimental.pallas import tpu_sc as plsc

B, H, D = 16384, 50, 32
NW = 32
BW = B // NW
GW = 128
K = BW // GW
NPAIR = H // 2

_mesh = plsc.VectorSubcoreMesh(core_axis_name="c", subcore_axis_name="s")


@functools.partial(
    pl.kernel,
    mesh=_mesh,
    out_type=jax.ShapeDtypeStruct((H, B, D), jnp.float32),
    compiler_params=pltpu.CompilerParams(
        needs_layout_passes=False, use_tc_tiling_on_sc=False
    ),
    scratch_types=[
        pltpu.VMEM((K, GW), jnp.int32),
        pltpu.VMEM((K, GW), jnp.int32),
        pltpu.VMEM((BW, D), jnp.float32),
        pltpu.VMEM((BW, D), jnp.float32),
        pltpu.SemaphoreType.DMA,
        pltpu.SemaphoreType.DMA,
        pltpu.SemaphoreType.DMA,
        pltpu.SemaphoreType.DMA,
    ],
)
def _emb(idx_hbm, table_hbm, out_hbm, idx_v0, idx_v1, rows_v0, rows_v1,
         sem_g0, sem_g1, sem_o0, sem_o1):
    wid = lax.axis_index("s") * 2 + lax.axis_index("c")
    idx_v = (idx_v0, idx_v1)
    rows_v = (rows_v0, rows_v1)
    sem_g = (sem_g0, sem_g1)
    sem_o = (sem_o0, sem_o1)

    def fire(b, h):
        pltpu.sync_copy(idx_hbm.at[h, pl.ds(wid * K, K)], idx_v[b])
        for j in range(K):
            pltpu.async_copy(
                table_hbm.at[idx_v[b].at[j]],
                rows_v[b].at[pl.ds(j * GW, GW)],
                sem_g[b],
            )

    def drain_gathers(b):
        for j in range(K):
            pltpu.make_async_copy(
                table_hbm.at[idx_v[b].at[j]],
                rows_v[b].at[pl.ds(j * GW, GW)],
                sem_g[b],
            ).wait()

    def out_fire(b, h):
        pltpu.async_copy(
            rows_v[b], out_hbm.at[h, pl.ds(wid * BW, BW)], sem_o[b]
        )

    def out_drain(b, h):
        pltpu.make_async_copy(
            rows_v[b], out_hbm.at[h, pl.ds(wid * BW, BW)], sem_o[b]
        ).wait()

    def process(b, h):
        def det_body(j, acc):
            a = acc
            for t in range(GW // 16):
                v = idx_v[b][j, pl.ds(t * 16, 16)]
                a = a + jnp.where(v == 0, 1, 0).astype(jnp.int32)
            return a

        acc = lax.fori_loop(0, K, det_body, jnp.zeros((16,), jnp.int32))
        any_pad = jnp.sum(acc) > 0

        drain_gathers(b)

        @pl.when(any_pad)
        def _mask_pass():
            def grp_body(g, carry2):
                vi = idx_v[b][g // (GW // 16), pl.ds((g % (GW // 16)) * 16, 16)]
                m = jnp.where(vi == 0, 0.0, 1.0).astype(jnp.float32)
                for r in range(16):
                    em = m.at[jnp.full((16,), r, jnp.int32)].get(
                        mode="promise_in_bounds"
                    )
                    row = g * 16 + r
                    for h2 in range(2):
                        cur = rows_v[b][row, pl.ds(h2 * 16, 16)]
                        rows_v[b][row, pl.ds(h2 * 16, 16)] = cur * em
                return carry2

            lax.fori_loop(0, BW // 16, grp_body, 0)

        out_fire(b, h)

    fire(0, 0)

    def pair_body(c2, carry):
        a = 2 * c2
        fire(1, a + 1)
        process(0, a)
        process(1, a + 1)
        out_drain(0, a)

        @pl.when(c2 < NPAIR - 1)
        def _():
            fire(0, a + 2)

        out_drain(1, a + 1)
        return carry

    lax.fori_loop(0, NPAIR, pair_body, 0)


def kernel(x, table):
    idx3d = jnp.transpose(x.astype(jnp.int32)).reshape(H, B // GW, GW)
    out_t = _emb(idx3d, table)
    return jnp.transpose(out_t, (1, 0, 2))
